# Initial kernel scaffold; baseline (speedup 1.0000x reference)
import jax
import jax.numpy as jnp
from jax.experimental import pallas as pl


def kernel(x, W1, b1, ln_gamma, ln_beta, W2, b2):
    raise NotImplementedError("write your pallas kernel here")


# speedup vs baseline: 1.6659x; 1.6659x over previous
import functools

import jax
import jax.numpy as jnp
from jax.experimental import pallas as pl

INPUT_DIM = 4096
NUM_EXPERTS = 64
TOP_K = 8
HIDDEN = 128
LN_EPS = 1e-5

BLOCK_T = 1024


def _fused_body(x_ref, w1_ref, b1_ref, g_ref, be_ref, w2_ref, b2t_ref,
                idx_ref, val_ref):
    h = jnp.dot(x_ref[...], w1_ref[...], preferred_element_type=jnp.float32)
    h = h + b1_ref[...]
    mean = jnp.mean(h, axis=-1, keepdims=True)
    var = jnp.mean(jnp.square(h - mean), axis=-1, keepdims=True)
    h = (h - mean) * jax.lax.rsqrt(var + LN_EPS) * g_ref[...] + be_ref[...]
    h = jnp.tanh(h)
    lt = jax.lax.dot_general(w2_ref[...], h, (((0,), (1,)), ((), ())),
                             preferred_element_type=jnp.float32)
    lt = lt + b2t_ref[...]
    m = jnp.max(lt, axis=0, keepdims=True)
    e = jnp.exp(lt - m)
    probs = e / jnp.sum(e, axis=0, keepdims=True)

    eidx = jax.lax.broadcasted_iota(jnp.int32, probs.shape, 0)
    work = probs
    idx_rows = []
    val_rows = []
    for _ in range(TOP_K):
        mx = jnp.max(work, axis=0, keepdims=True)
        amx = jnp.min(jnp.where(work == mx, eidx, NUM_EXPERTS),
                      axis=0, keepdims=True)
        idx_rows.append(amx)
        val_rows.append(mx)
        work = jnp.where(eidx == amx, -1.0, work)
    idx_ref[...] = jnp.concatenate(idx_rows, axis=0).T
    val_ref[...] = jnp.concatenate(val_rows, axis=0).T


@functools.partial(jax.jit, static_argnames=())
def kernel(x, W1, b1, ln_gamma, ln_beta, W2, b2):
    tokens = x.shape[0]
    grid = (tokens // BLOCK_T,)
    b1 = b1.reshape(1, HIDDEN)
    ln_gamma = ln_gamma.reshape(1, HIDDEN)
    ln_beta = ln_beta.reshape(1, HIDDEN)
    b2t = b2.reshape(NUM_EXPERTS, 1)
    idx, vals = pl.pallas_call(
        _fused_body,
        grid=grid,
        in_specs=[
            pl.BlockSpec((BLOCK_T, INPUT_DIM), lambda i: (i, 0)),
            pl.BlockSpec((INPUT_DIM, HIDDEN), lambda i: (0, 0)),
            pl.BlockSpec((1, HIDDEN), lambda i: (0, 0)),
            pl.BlockSpec((1, HIDDEN), lambda i: (0, 0)),
            pl.BlockSpec((1, HIDDEN), lambda i: (0, 0)),
            pl.BlockSpec((HIDDEN, NUM_EXPERTS), lambda i: (0, 0)),
            pl.BlockSpec((NUM_EXPERTS, 1), lambda i: (0, 0)),
        ],
        out_specs=[
            pl.BlockSpec((BLOCK_T, TOP_K), lambda i: (i, 0)),
            pl.BlockSpec((BLOCK_T, TOP_K), lambda i: (i, 0)),
        ],
        out_shape=[
            jax.ShapeDtypeStruct((tokens, TOP_K), jnp.int32),
            jax.ShapeDtypeStruct((tokens, TOP_K), jnp.float32),
        ],
    )(x, W1, b1, ln_gamma, ln_beta, W2, b2t)
    return idx, vals
